# Initial kernel scaffold; baseline (speedup 1.0000x reference)
import jax
import jax.numpy as jnp
from jax.experimental import pallas as pl


def kernel(indices, weight):
    raise NotImplementedError("write your pallas kernel here")


# speedup vs baseline: 2.4324x; 2.4324x over previous
import functools

import jax
import jax.numpy as jnp
from jax import lax
from jax.experimental import pallas as pl
from jax.experimental.pallas import tpu as pltpu
from jax.experimental.pallas import tpu_sc as plsc

BATCH = 16384
HIST = 50
DIM = 64
NUM_WORKERS = 32
BAGS_PER_WORKER = BATCH // NUM_WORKERS
CHUNK_BAGS = 16
ROWS_PER_CHUNK = CHUNK_BAGS * HIST
NUM_CHUNKS = BAGS_PER_WORKER // CHUNK_BAGS
LANES = 16
DSUB = DIM // LANES


def _sc_bag_mean(flat_idx, weight):
    mesh = plsc.VectorSubcoreMesh(core_axis_name="c", subcore_axis_name="s")

    @functools.partial(
        pl.kernel,
        mesh=mesh,
        compiler_params=pltpu.CompilerParams(use_tc_tiling_on_sc=False),
        out_type=jax.ShapeDtypeStruct((BATCH, DIM), jnp.float32),
        scratch_types=[
            pltpu.VMEM((ROWS_PER_CHUNK,), jnp.int32),
            pltpu.VMEM((ROWS_PER_CHUNK, DIM), jnp.float32),
            pltpu.VMEM((CHUNK_BAGS, DIM), jnp.float32),
            pltpu.SemaphoreType.DMA,
        ],
    )
    def k(idx_hbm, w_hbm, out_hbm, idx_v, rows_v, acc_v, sem):
        wid = lax.axis_index("s") * 2 + lax.axis_index("c")
        bag_base = wid * BAGS_PER_WORKER
        scale = jnp.full((LANES,), 1.0 / HIST, jnp.float32)

        def chunk_body(chunk, _):
            first_bag = bag_base + chunk * CHUNK_BAGS
            pltpu.sync_copy(idx_hbm.at[pl.ds(first_bag * HIST, ROWS_PER_CHUNK)],
                            idx_v)
            pltpu.async_copy(w_hbm.at[idx_v], rows_v, sem).wait()

            def bag_body(c, _):
                base_row = c * HIST
                accs = [jnp.zeros((LANES,), jnp.float32) for _ in range(DSUB)]
                for r in range(HIST):
                    for j in range(DSUB):
                        accs[j] = accs[j] + rows_v[base_row + r,
                                                   pl.ds(j * LANES, LANES)]
                for j in range(DSUB):
                    acc_v[c, pl.ds(j * LANES, LANES)] = accs[j] * scale
                return ()

            lax.fori_loop(0, CHUNK_BAGS, bag_body, ())
            pltpu.sync_copy(acc_v, out_hbm.at[pl.ds(first_bag, CHUNK_BAGS)])
            return ()

        lax.fori_loop(0, NUM_CHUNKS, chunk_body, ())

    return k(flat_idx, weight)


def kernel(indices, weight):
    flat_idx = indices.reshape(-1).astype(jnp.int32)
    return _sc_bag_mean(flat_idx, weight)


# speedup vs baseline: 2.6846x; 1.1037x over previous
import functools

import jax
import jax.numpy as jnp
from jax import lax
from jax.experimental import pallas as pl
from jax.experimental.pallas import tpu as pltpu
from jax.experimental.pallas import tpu_sc as plsc

BATCH = 16384
HIST = 50
DIM = 64
NUM_WORKERS = 32
BAGS_PER_WORKER = BATCH // NUM_WORKERS
CHUNK_BAGS = 16
ROWS_PER_CHUNK = CHUNK_BAGS * HIST
NUM_CHUNKS = BAGS_PER_WORKER // CHUNK_BAGS
LANES = 16
DSUB = DIM // LANES


def _sc_bag_mean(flat_idx, weight):
    mesh = plsc.VectorSubcoreMesh(core_axis_name="c", subcore_axis_name="s")

    @functools.partial(
        pl.kernel,
        mesh=mesh,
        compiler_params=pltpu.CompilerParams(use_tc_tiling_on_sc=False),
        out_type=jax.ShapeDtypeStruct((BATCH, DIM), jnp.float32),
        scratch_types=[
            pltpu.VMEM((2, ROWS_PER_CHUNK), jnp.int32),
            pltpu.VMEM((2, ROWS_PER_CHUNK, DIM), jnp.float32),
            pltpu.VMEM((CHUNK_BAGS, DIM), jnp.float32),
            pltpu.SemaphoreType.DMA,
            pltpu.SemaphoreType.DMA,
        ],
    )
    def k(idx_hbm, w_hbm, out_hbm, idx_v, rows_v, acc_v, sem0, sem1):
        wid = lax.axis_index("s") * 2 + lax.axis_index("c")
        bag_base = wid * BAGS_PER_WORKER
        scale = jnp.full((LANES,), 1.0 / HIST, jnp.float32)
        sems = (sem0, sem1)

        def start_gather(chunk, buf, sem):
            first_bag = bag_base + chunk * CHUNK_BAGS
            pltpu.sync_copy(
                idx_hbm.at[pl.ds(first_bag * HIST, ROWS_PER_CHUNK)],
                idx_v.at[buf])
            pltpu.async_copy(w_hbm.at[idx_v.at[buf]], rows_v.at[buf], sem)

        def finish_chunk(chunk, buf, sem):
            pltpu.make_async_copy(
                w_hbm.at[idx_v.at[buf]], rows_v.at[buf], sem).wait()

            def bag_body(c, _):
                base_row = c * HIST
                accs = [jnp.zeros((LANES,), jnp.float32) for _ in range(DSUB)]
                for r in range(HIST):
                    for j in range(DSUB):
                        accs[j] = accs[j] + rows_v[buf, base_row + r,
                                                   pl.ds(j * LANES, LANES)]
                for j in range(DSUB):
                    acc_v[c, pl.ds(j * LANES, LANES)] = accs[j] * scale
                return ()

            lax.fori_loop(0, CHUNK_BAGS, bag_body, ())
            first_bag = bag_base + chunk * CHUNK_BAGS
            pltpu.sync_copy(acc_v, out_hbm.at[pl.ds(first_bag, CHUNK_BAGS)])

        start_gather(0, 0, sems[0])

        def pair_body(p, _):
            c0 = 2 * p
            start_gather(c0 + 1, 1, sems[1])
            finish_chunk(c0, 0, sems[0])

            @pl.when(p < NUM_CHUNKS // 2 - 1)
            def _():
                start_gather(c0 + 2, 0, sems[0])

            finish_chunk(c0 + 1, 1, sems[1])
            return ()

        lax.fori_loop(0, NUM_CHUNKS // 2, pair_body, ())

    return k(flat_idx, weight)


def kernel(indices, weight):
    flat_idx = indices.reshape(-1).astype(jnp.int32)
    return _sc_bag_mean(flat_idx, weight)
